# baseline (speedup 1.0000x reference)
import functools
import math

import jax
import jax.numpy as jnp
from jax import lax
from jax.experimental import pallas as pl
from jax.experimental.pallas import tpu as pltpu
from jax.experimental.pallas import tpu_sc as plsc

_D = 32
_L = 16
_NC = 2
_NS = 16
_NW = _NC * _NS

_BL = 128
_HC = 4
_NBUF = 3


def _emb_kernel(n_batch, n_hist, voc):
    n_chunks = n_hist // _HC
    n_groups = (n_chunks + _NBUF - 1) // _NBUF
    scale = math.sqrt(float(_D))
    mesh = plsc.VectorSubcoreMesh(core_axis_name="c", subcore_axis_name="s")

    @functools.partial(
        pl.kernel,
        mesh=mesh,
        out_type=jax.ShapeDtypeStruct(
            (n_hist, _D // 8, n_batch // _BL, 8, _BL), jnp.float32
        ),
        scratch_types=[
            pltpu.VMEM((_NBUF, _HC, _BL), jnp.int32),
            pltpu.VMEM((_NBUF, _HC * _BL, _D), jnp.float32),
            pltpu.VMEM((_NBUF, _HC, _D // 8, 8, _BL), jnp.float32),
            pltpu.SemaphoreType.DMA((_NBUF,)),
            pltpu.SemaphoreType.DMA((_NBUF,)),
        ],
        compiler_params=pltpu.CompilerParams(
            use_tc_tiling_on_sc=False, needs_layout_passes=False
        ),
    )
    def k(xt_hbm, tbl_hbm, out_hbm, idx_v, rows_v, obuf_v, gsem, osem):
        wid = lax.axis_index("s") * _NC + lax.axis_index("c")
        b0 = wid * _BL

        def gathers(b):
            return [
                pltpu.make_async_copy(
                    tbl_hbm.at[idx_v.at[b].at[j]],
                    rows_v.at[b].at[pl.ds(j * _BL, _BL)],
                    gsem.at[b],
                )
                for j in range(_HC)
            ]

        def out_copy(ci, b):
            return pltpu.make_async_copy(
                obuf_v.at[b],
                out_hbm.at[pl.ds(ci * _HC, _HC), :, wid],
                osem.at[b],
            )

        def issue(ci, b):
            pltpu.sync_copy(
                xt_hbm.at[pl.ds(ci * _HC, _HC), pl.ds(b0, _BL)], idx_v.at[b]
            )
            for c in gathers(b):
                c.start()

        for b in range(_NBUF - 1):
            issue(b, b)

        lane = lax.iota(jnp.int32, _L)
        rivs = [
            [lane + (j * _BL + k * _L) for k in range(_BL // _L)]
            for j in range(_HC)
        ]

        def step(ci, b):
            bp = (b + _NBUF - 1) % _NBUF

            @pl.when(ci >= 1)
            def _():
                out_copy(ci - 1, bp).wait()

            @pl.when(ci + _NBUF - 1 < n_chunks)
            def _():
                issue(ci + _NBUF - 1, bp)

            for c in gathers(b):
                c.wait()

            def dpass(d, _):
                dv = lane * 0 + d
                dc = d >> 3
                dr = d & 7
                for j in range(_HC):
                    for kk in range(_BL // _L):
                        v = plsc.load_gather(
                            rows_v.at[b], [rivs[j][kk], dv]
                        )
                        obuf_v[b, j, dc, dr, pl.ds(kk * _L, _L)] = v * scale
                return 0

            lax.fori_loop(0, _D, dpass, 0)
            out_copy(ci, b).start()

        def group(cj, _):
            for b in range(_NBUF):
                ci = cj * _NBUF + b

                @pl.when(ci < n_chunks)
                def _():
                    step(ci, b)

            return 0

        lax.fori_loop(0, n_groups, group, 0)
        out_copy(n_chunks - 1, (n_chunks - 1) % _NBUF).wait()

    return k


def kernel(x, W):
    b, h = x.shape
    voc = W.shape[0]
    xt = x.T
    out5 = _emb_kernel(b, h, voc)(xt, W)
    return jnp.transpose(out5, (2, 4, 0, 1, 3)).reshape(b, h, _D)


# speedup vs baseline: 1.0970x; 1.0970x over previous
import functools
import math

import jax
import jax.numpy as jnp
from jax import lax
from jax.experimental import pallas as pl
from jax.experimental.pallas import tpu as pltpu
from jax.experimental.pallas import tpu_sc as plsc

_D = 32
_L = 16
_NC = 2
_NS = 16
_NW = _NC * _NS

_BL = 128
_HC = 4
_NBUF = 3
_SEG = 8 * _BL


def _emb_kernel(n_batch, n_hist, voc):
    n_chunks = n_hist // _HC
    n_groups = (n_chunks + _NBUF - 1) // _NBUF
    scale = math.sqrt(float(_D))
    ndc = _D // 8
    mesh = plsc.VectorSubcoreMesh(core_axis_name="c", subcore_axis_name="s")

    @functools.partial(
        pl.kernel,
        mesh=mesh,
        out_type=jax.ShapeDtypeStruct((n_hist, n_batch * _D), jnp.float32),
        scratch_types=[
            pltpu.VMEM((_NBUF, _HC, _BL), jnp.int32),
            pltpu.VMEM((_NBUF, _HC * _BL, _D), jnp.float32),
            pltpu.VMEM((_NBUF, _HC, _D * _BL), jnp.float32),
            pltpu.SemaphoreType.DMA((_NBUF,)),
            pltpu.SemaphoreType.DMA((_NBUF,)),
        ],
        compiler_params=pltpu.CompilerParams(
            use_tc_tiling_on_sc=False, needs_layout_passes=False
        ),
    )
    def k(xt_hbm, tbl_hbm, out_hbm, idx_v, rows_v, obuf_v, gsem, osem):
        wid = lax.axis_index("s") * _NC + lax.axis_index("c")
        b0 = wid * _BL

        def gathers(b):
            return [
                pltpu.make_async_copy(
                    tbl_hbm.at[idx_v.at[b].at[j]],
                    rows_v.at[b].at[pl.ds(j * _BL, _BL)],
                    gsem.at[b],
                )
                for j in range(_HC)
            ]

        def out_copies(ci, b):
            return [
                pltpu.make_async_copy(
                    obuf_v.at[b].at[:, pl.ds(dc * _SEG, _SEG)],
                    out_hbm.at[
                        pl.ds(ci * _HC, _HC),
                        pl.ds(dc * (n_batch * 8) + wid * _SEG, _SEG),
                    ],
                    osem.at[b],
                )
                for dc in range(ndc)
            ]

        def issue(ci, b):
            pltpu.sync_copy(
                xt_hbm.at[pl.ds(ci * _HC, _HC), pl.ds(b0, _BL)], idx_v.at[b]
            )
            for c in gathers(b):
                c.start()

        for b in range(_NBUF - 1):
            issue(b, b)

        lane = lax.iota(jnp.int32, _L)
        ilo0 = lane * _BL
        ihi0 = lane * _BL + _L * _BL

        def step(ci, b):
            bp = (b + _NBUF - 1) % _NBUF

            @pl.when(ci >= 1)
            def _():
                for c in out_copies(ci - 1, bp):
                    c.wait()

            @pl.when(ci + _NBUF - 1 < n_chunks)
            def _():
                issue(ci + _NBUF - 1, bp)

            for c in gathers(b):
                c.wait()

            for j in range(_HC):
                dst = obuf_v.at[b].at[j]

                def tbody(br, carry):
                    ilo, ihi = carry
                    r = j * _BL + br
                    plsc.store_scatter(
                        dst, [ilo], rows_v[b, r, pl.ds(0, _L)] * scale
                    )
                    plsc.store_scatter(
                        dst, [ihi], rows_v[b, r, pl.ds(_L, _L)] * scale
                    )
                    return ilo + 1, ihi + 1

                lax.fori_loop(0, _BL, tbody, (ilo0, ihi0), unroll=4)
            for c in out_copies(ci, b):
                c.start()

        def group(cj, _):
            for b in range(_NBUF):
                ci = cj * _NBUF + b

                @pl.when(ci < n_chunks)
                def _():
                    step(ci, b)

            return 0

        lax.fori_loop(0, n_groups, group, 0)
        for c in out_copies(n_chunks - 1, (n_chunks - 1) % _NBUF):
            c.wait()

    return k


def kernel(x, W):
    b, h = x.shape
    voc = W.shape[0]
    xt = x.T
    out2 = _emb_kernel(b, h, voc)(xt, W)
    out5 = out2.reshape(h, _D // 8, b // _BL, 8, _BL)
    return jnp.transpose(out5, (2, 4, 0, 1, 3)).reshape(b, h, _D)


# speedup vs baseline: 1.3452x; 1.2262x over previous
import functools
import math

import jax
import jax.numpy as jnp
from jax import lax
from jax.experimental import pallas as pl
from jax.experimental.pallas import tpu as pltpu
from jax.experimental.pallas import tpu_sc as plsc

_D = 32
_L = 16
_NC = 2
_NS = 16
_NW = _NC * _NS

_BL = 128
_HC = 4
_NBUF = 3
_SEG = 8 * _BL


def _emb_kernel(n_batch, n_hist, voc):
    n_chunks = n_hist // _HC
    n_groups = (n_chunks + _NBUF - 1) // _NBUF
    scale = math.sqrt(float(_D))
    ndc = _D // 8
    mesh = plsc.VectorSubcoreMesh(core_axis_name="c", subcore_axis_name="s")

    @functools.partial(
        pl.kernel,
        mesh=mesh,
        out_type=jax.ShapeDtypeStruct((n_hist, n_batch * _D), jnp.float32),
        scratch_types=[
            pltpu.VMEM((_NBUF, _HC, _BL), jnp.int32),
            pltpu.VMEM((_NBUF, _HC * _BL, _D), jnp.float32),
            pltpu.VMEM((_NBUF, _HC, _D * _BL), jnp.float32),
            pltpu.SemaphoreType.DMA((_NBUF,)),
            pltpu.SemaphoreType.DMA((_NBUF,)),
        ],
        compiler_params=pltpu.CompilerParams(
            use_tc_tiling_on_sc=False, needs_layout_passes=False
        ),
    )
    def k(xt_hbm, tbl_hbm, out_hbm, idx_v, rows_v, obuf_v, gsem, osem):
        wid = lax.axis_index("s") * _NC + lax.axis_index("c")
        b0 = wid * _BL

        def gathers(b):
            return [
                pltpu.make_async_copy(
                    tbl_hbm.at[idx_v.at[b].at[j]],
                    rows_v.at[b].at[pl.ds(j * _BL, _BL)],
                    gsem.at[b],
                )
                for j in range(_HC)
            ]

        def out_copies(ci, b):
            return [
                pltpu.make_async_copy(
                    obuf_v.at[b].at[:, pl.ds(dc * _SEG, _SEG)],
                    out_hbm.at[
                        pl.ds(ci * _HC, _HC),
                        pl.ds(dc * (n_batch * 8) + wid * _SEG, _SEG),
                    ],
                    osem.at[b],
                )
                for dc in range(ndc)
            ]

        def issue(ci, b):
            pltpu.sync_copy(
                xt_hbm.at[pl.ds(ci * _HC, _HC), pl.ds(b0, _BL)], idx_v.at[b]
            )
            for c in gathers(b):
                c.start()

        for b in range(_NBUF - 1):
            issue(b, b)

        lane = lax.iota(jnp.int32, _L)
        ilo0 = lane * _BL
        ihi0 = lane * _BL + _L * _BL

        def step(ci, b):
            bp = (b + _NBUF - 1) % _NBUF

            @pl.when(ci >= 1)
            def _():
                for c in out_copies(ci - 1, bp):
                    c.wait()

            @pl.when(ci + _NBUF - 1 < n_chunks)
            def _():
                issue(ci + _NBUF - 1, bp)

            for c in gathers(b):
                c.wait()

            for j in range(_HC):
                dst = obuf_v.at[b].at[j]

                @plsc.parallel_loop(0, _BL, unroll=8)
                def _(br):
                    r = j * _BL + br
                    plsc.store_scatter(
                        dst, [ilo0 + br], rows_v[b, r, pl.ds(0, _L)] * scale
                    )
                    plsc.store_scatter(
                        dst, [ihi0 + br], rows_v[b, r, pl.ds(_L, _L)] * scale
                    )
            for c in out_copies(ci, b):
                c.start()

        def group(cj, _):
            for b in range(_NBUF):
                ci = cj * _NBUF + b

                @pl.when(ci < n_chunks)
                def _():
                    step(ci, b)

            return 0

        lax.fori_loop(0, n_groups, group, 0)
        for c in out_copies(n_chunks - 1, (n_chunks - 1) % _NBUF):
            c.wait()

    return k


def kernel(x, W):
    b, h = x.shape
    voc = W.shape[0]
    xt = x.T
    out2 = _emb_kernel(b, h, voc)(xt, W)
    out5 = out2.reshape(h, _D // 8, b // _BL, 8, _BL)
    return jnp.transpose(out5, (2, 4, 0, 1, 3)).reshape(b, h, _D)
